# baseline (speedup 1.0000x reference)
import functools

import jax
import jax.numpy as jnp
from jax import lax
from jax.experimental import pallas as pl
from jax.experimental.pallas import tpu as pltpu
from jax.experimental.pallas import tpu_sc as plsc

NC, NS = 2, 16
NW = NC * NS


def _sc_mesh():
    return plsc.VectorSubcoreMesh(
        core_axis_name="c", subcore_axis_name="s", num_cores=NC, num_subcores=NS
    )


def _make_gather(E, N, D, CH):
    EW = E // NW
    n_chunks = EW // CH

    @functools.partial(
        pl.kernel,
        out_type=(
            jax.ShapeDtypeStruct((E, D), jnp.float32),
            jax.ShapeDtypeStruct((E, D), jnp.float32),
        ),
        mesh=_sc_mesh(),
        scratch_types=[
            pltpu.VMEM((EW,), jnp.int32),
            pltpu.VMEM((EW,), jnp.int32),
            pltpu.VMEM((CH, D), jnp.float32),
            pltpu.VMEM((CH, D), jnp.float32),
            pltpu.SemaphoreType.DMA,
            pltpu.SemaphoreType.DMA,
        ],
    )
    def gather_k(x_hbm, dst_hbm, src_hbm, xi_hbm, xj_hbm,
                 idx_i, idx_j, rows_i, rows_j, sem_i, sem_j):
        wid = lax.axis_index("s") * NC + lax.axis_index("c")
        base = wid * EW
        pltpu.sync_copy(dst_hbm.at[pl.ds(base, EW)], idx_i)
        pltpu.sync_copy(src_hbm.at[pl.ds(base, EW)], idx_j)

        def body(ci, _):
            off = ci * CH
            cp_i = pltpu.async_copy(
                x_hbm.at[idx_i.at[pl.ds(off, CH)]], rows_i, sem_i)
            cp_j = pltpu.async_copy(
                x_hbm.at[idx_j.at[pl.ds(off, CH)]], rows_j, sem_j)
            cp_i.wait()
            pltpu.sync_copy(rows_i, xi_hbm.at[pl.ds(base + off, CH)])
            cp_j.wait()
            pltpu.sync_copy(rows_j, xj_hbm.at[pl.ds(base + off, CH)])
            return _

        lax.fori_loop(0, n_chunks, body, None)

    return gather_k


def _make_scatter(E, N, D, CH):
    EW = E // NW
    n_chunks = EW // CH

    @functools.partial(
        pl.kernel,
        out_type=jax.ShapeDtypeStruct((NC, N, D), jnp.float32),
        mesh=_sc_mesh(),
        scratch_types=[
            pltpu.VMEM((EW,), jnp.int32),
            pltpu.VMEM((CH, D), jnp.float32),
            pltpu.VMEM_SHARED((N, D), jnp.float32),
            pltpu.SemaphoreType.DMA,
        ],
    )
    def scatter_k(msg_hbm, dst_hbm, zeros_hbm, out_hbm,
                  idx_v, rows_v, aggr_sh, sem):
        c = lax.axis_index("c")
        s = lax.axis_index("s")
        wid = s * NC + c
        base = wid * EW

        @pl.when(s == 0)
        def _():
            pltpu.sync_copy(zeros_hbm, aggr_sh)

        plsc.subcore_barrier()
        pltpu.sync_copy(dst_hbm.at[pl.ds(base, EW)], idx_v)

        def body(ci, _):
            off = ci * CH
            pltpu.sync_copy(msg_hbm.at[pl.ds(base + off, CH)], rows_v)
            pltpu.sync_copy(rows_v, aggr_sh.at[idx_v.at[pl.ds(off, CH)]],
                            add=True)
            return _

        lax.fori_loop(0, n_chunks, body, None)
        plsc.subcore_barrier()

        @pl.when(s == 0)
        def _():
            pltpu.sync_copy(aggr_sh, out_hbm.at[c])

    return scatter_k


def _edge_mlp_body(xi_ref, xj_ref, ea_ref,
                   w1a_ref, w1b_ref, w1c_ref, b1_ref,
                   w2_ref, b2_ref, w3_ref, b3_ref, w4_ref, b4_ref,
                   out_ref):
    bf = jnp.bfloat16
    h = (jnp.dot(xi_ref[...].astype(bf), w1a_ref[...], preferred_element_type=jnp.float32)
         + jnp.dot(xj_ref[...].astype(bf), w1b_ref[...], preferred_element_type=jnp.float32)
         + jnp.dot(ea_ref[...], w1c_ref[...], preferred_element_type=jnp.float32)
         + b1_ref[...])
    h = jnp.maximum(h, 0.0).astype(bf)
    h = jnp.maximum(
        jnp.dot(h, w2_ref[...], preferred_element_type=jnp.float32) + b2_ref[...], 0.0).astype(bf)
    h = jnp.maximum(
        jnp.dot(h, w3_ref[...], preferred_element_type=jnp.float32) + b3_ref[...], 0.0).astype(bf)
    out_ref[...] = (
        jnp.dot(h, w4_ref[...], preferred_element_type=jnp.float32) + b4_ref[...])


def _run_edge_mlp(xi, xj, ea, w1a, w1b, w1c, b1, w2, b2, w3, b3, w4, b4, BE):
    E, D = xi.shape
    DE = ea.shape[1]
    H = w2.shape[0]
    M = w4.shape[1]
    nblk = E // BE
    full = lambda shape: pl.BlockSpec(shape, lambda i: (0,) * len(shape))
    return pl.pallas_call(
        _edge_mlp_body,
        grid=(nblk,),
        in_specs=[
            pl.BlockSpec((BE, D), lambda i: (i, 0)),
            pl.BlockSpec((BE, D), lambda i: (i, 0)),
            pl.BlockSpec((BE, DE), lambda i: (i, 0)),
            full((D, H)), full((D, H)), full((DE, H)), full((1, H)),
            full((H, H)), full((1, H)),
            full((H, H)), full((1, H)),
            full((H, M)), full((1, M)),
        ],
        out_specs=pl.BlockSpec((BE, M), lambda i: (i, 0)),
        out_shape=jax.ShapeDtypeStruct((E, M), jnp.float32),
    )(xi, xj, ea, w1a, w1b, w1c, b1, w2, b2, w3, b3, w4, b4)


def _node_pool_body(x_ref, ap_ref, batch_ref,
                    v1a_ref, v1b_ref, c1_ref, v2_ref, c2_ref,
                    v3_ref, c3_ref, v4_ref, c4_ref, wl_ref, bl_ref,
                    out_ref, sum_acc, cnt_acc, *, nblk, n_graphs):
    i = pl.program_id(0)

    @pl.when(i == 0)
    def _():
        sum_acc[...] = jnp.zeros_like(sum_acc)
        cnt_acc[...] = jnp.zeros_like(cnt_acc)

    bf = jnp.bfloat16
    aggr = (ap_ref[0] + ap_ref[1]).astype(bf)
    h = (jnp.dot(x_ref[...].astype(bf), v1a_ref[...], preferred_element_type=jnp.float32)
         + jnp.dot(aggr, v1b_ref[...], preferred_element_type=jnp.float32)
         + c1_ref[...])
    h = jnp.maximum(h, 0.0).astype(bf)
    h = jnp.maximum(
        jnp.dot(h, v2_ref[...], preferred_element_type=jnp.float32) + c2_ref[...], 0.0).astype(bf)
    h = jnp.maximum(
        jnp.dot(h, v3_ref[...], preferred_element_type=jnp.float32) + c3_ref[...], 0.0).astype(bf)
    node = (jnp.dot(h, v4_ref[...], preferred_element_type=jnp.float32)
            + c4_ref[...])

    b = batch_ref[0]
    gids = lax.broadcasted_iota(jnp.int32, (n_graphs, b.shape[1]), 0)
    mask = (gids == b).astype(jnp.float32)
    sum_acc[...] += jnp.dot(mask, node, preferred_element_type=jnp.float32)
    cnt_acc[...] += jnp.sum(mask, axis=1, keepdims=True)

    @pl.when(i == nblk - 1)
    def _():
        pooled = sum_acc[...] / jnp.maximum(cnt_acc[...], 1.0)
        out_ref[...] = (
            jnp.dot(pooled, wl_ref[...], preferred_element_type=jnp.float32)
            + bl_ref[...])


def _run_node_pool(x, aggr_p, batch3, v1a, v1b, c1, v2, c2, v3, c3, v4, c4,
                   wl, bl, BN, n_graphs):
    N, D = x.shape
    H = v2.shape[0]
    NH = v4.shape[1]
    P = wl.shape[1]
    nblk = N // BN
    full = lambda shape: pl.BlockSpec(shape, lambda i: (0,) * len(shape))
    body = functools.partial(_node_pool_body, nblk=nblk, n_graphs=n_graphs)
    return pl.pallas_call(
        body,
        grid=(nblk,),
        in_specs=[
            pl.BlockSpec((BN, D), lambda i: (i, 0)),
            pl.BlockSpec((NC, BN, D), lambda i: (0, i, 0)),
            pl.BlockSpec((1, 1, BN), lambda i: (i, 0, 0)),
            full((D, H)), full((D, H)), full((1, H)),
            full((H, H)), full((1, H)),
            full((H, H)), full((1, H)),
            full((H, NH)), full((1, NH)),
            full((NH, P)), full((1, P)),
        ],
        out_specs=pl.BlockSpec((n_graphs, P), lambda i: (0, 0)),
        out_shape=jax.ShapeDtypeStruct((n_graphs, P), jnp.float32),
        scratch_shapes=[
            pltpu.VMEM((n_graphs, NH), jnp.float32),
            pltpu.VMEM((n_graphs, 1), jnp.float32),
        ],
    )(x, aggr_p, batch3, v1a, v1b, c1, v2, c2, v3, c3, v4, c4, wl, bl)


def kernel(x, edge_index, edge_attr, batch,
           W1, b1, W2, b2, W3, b3, W4, b4,
           V1, c1, V2, c2, V3, c3, V4, c4,
           Wl, bl):
    N, D = x.shape
    E = edge_index.shape[1]
    DE = edge_attr.shape[1]
    N_GRAPHS = 64
    CH = 80
    BE = 1280
    BN = 1000

    src = edge_index[0]
    dst = edge_index[1]

    xi, xj = _make_gather(E, N, D, CH)(x, dst, src)

    bf = jnp.bfloat16
    w1a, w1b, w1c = W1[:D].astype(bf), W1[D:2 * D].astype(bf), W1[2 * D:].astype(bf)
    msg = _run_edge_mlp(
        xi, xj, edge_attr.astype(bf),
        w1a, w1b, w1c, b1.reshape(1, -1),
        W2.astype(bf), b2.reshape(1, -1), W3.astype(bf), b3.reshape(1, -1),
        W4.astype(bf), b4.reshape(1, -1),
        BE)

    zeros = jnp.zeros((N, D), jnp.float32)
    aggr_p = _make_scatter(E, N, D, CH)(msg, dst, zeros)

    batch3 = batch.reshape(N // BN, 1, BN)
    v1a, v1b = V1[:D].astype(bf), V1[D:].astype(bf)
    out = _run_node_pool(
        x, aggr_p, batch3,
        v1a, v1b, c1.reshape(1, -1),
        V2.astype(bf), c2.reshape(1, -1), V3.astype(bf), c3.reshape(1, -1),
        V4.astype(bf), c4.reshape(1, -1),
        Wl, bl.reshape(1, -1),
        BN, N_GRAPHS)
    return out


# speedup vs baseline: 1.1505x; 1.1505x over previous
import functools

import jax
import jax.numpy as jnp
from jax import lax
from jax.experimental import pallas as pl
from jax.experimental.pallas import tpu as pltpu
from jax.experimental.pallas import tpu_sc as plsc

NC, NS = 2, 16
NW = NC * NS


def _sc_mesh():
    return plsc.VectorSubcoreMesh(
        core_axis_name="c", subcore_axis_name="s", num_cores=NC, num_subcores=NS
    )


def _make_gather(E, N, D, SB, dtype):
    EW = E // NW
    n_sup = EW // SB
    assert EW % SB == 0 and SB % 8 == 0
    sub = []
    off = 0
    while off < SB:
        c = min(120, SB - off)
        sub.append((off, c))
        off += c
    assert all(c % 8 == 0 and o % 8 == 0 for o, c in sub)

    @functools.partial(
        pl.kernel,
        out_type=(
            jax.ShapeDtypeStruct((E, D), dtype),
            jax.ShapeDtypeStruct((E, D), dtype),
        ),
        mesh=_sc_mesh(),
        scratch_types=[
            pltpu.VMEM((EW,), jnp.int32),
            pltpu.VMEM((EW,), jnp.int32),
            pltpu.VMEM((2, SB, D), dtype),
            pltpu.VMEM((2, SB, D), dtype),
            pltpu.SemaphoreType.DMA,
            pltpu.SemaphoreType.DMA,
        ],
    )
    def gather_k(x_hbm, dst_hbm, src_hbm, xi_hbm, xj_hbm,
                 idx_i, idx_j, rows_i, rows_j, sem0, sem1):
        wid = lax.axis_index("s") * NC + lax.axis_index("c")
        base = wid * EW
        pltpu.sync_copy(dst_hbm.at[pl.ds(base, EW)], idx_i)
        pltpu.sync_copy(src_hbm.at[pl.ds(base, EW)], idx_j)
        sems = (sem0, sem1)

        def issue(si, b):
            soff = si * SB
            for o, c in sub:
                pltpu.async_copy(
                    x_hbm.at[idx_i.at[pl.ds(soff + o, c)]],
                    rows_i.at[b].at[pl.ds(o, c)], sems[b])
                pltpu.async_copy(
                    x_hbm.at[idx_j.at[pl.ds(soff + o, c)]],
                    rows_j.at[b].at[pl.ds(o, c)], sems[b])

        def drain(si, b):
            soff = si * SB
            for o, c in sub:
                pltpu.make_async_copy(
                    x_hbm.at[idx_i.at[pl.ds(soff + o, c)]],
                    rows_i.at[b].at[pl.ds(o, c)], sems[b]).wait()
                pltpu.make_async_copy(
                    x_hbm.at[idx_j.at[pl.ds(soff + o, c)]],
                    rows_j.at[b].at[pl.ds(o, c)], sems[b]).wait()

        issue(0, 0)

        def outer(oi, _):
            for b in range(2):
                si = 2 * oi + b

                @pl.when(si + 1 < n_sup)
                def _():
                    issue(si + 1, 1 - b)

                drain(si, b)
                pltpu.sync_copy(rows_i.at[b], xi_hbm.at[pl.ds(base + si * SB, SB)])
                pltpu.sync_copy(rows_j.at[b], xj_hbm.at[pl.ds(base + si * SB, SB)])
            return _

        lax.fori_loop(0, n_sup // 2, outer, None)

    return gather_k


def _make_scatter(E, N, D, SB):
    EW = E // NW
    n_sup = EW // SB
    assert EW % SB == 0 and SB % 8 == 0
    sub = []
    off = 0
    while off < SB:
        c = min(120, SB - off)
        sub.append((off, c))
        off += c

    @functools.partial(
        pl.kernel,
        out_type=jax.ShapeDtypeStruct((NC, N, D), jnp.float32),
        mesh=_sc_mesh(),
        scratch_types=[
            pltpu.VMEM((EW,), jnp.int32),
            pltpu.VMEM((2, SB, D), jnp.float32),
            pltpu.VMEM_SHARED((N, D), jnp.float32),
            pltpu.SemaphoreType.DMA,
            pltpu.SemaphoreType.DMA,
        ],
    )
    def scatter_k(msg_hbm, dst_hbm, zeros_hbm, out_hbm,
                  idx_v, rows_v, aggr_sh, sem0, sem1):
        c = lax.axis_index("c")
        s = lax.axis_index("s")
        wid = s * NC + c
        base = wid * EW
        sems = (sem0, sem1)

        @pl.when(s == 0)
        def _():
            pltpu.sync_copy(zeros_hbm, aggr_sh)

        plsc.subcore_barrier()
        pltpu.sync_copy(dst_hbm.at[pl.ds(base, EW)], idx_v)

        def issue_add(si, b):
            soff = si * SB
            for o, cc in sub:
                pltpu.async_copy(
                    rows_v.at[b].at[pl.ds(o, cc)],
                    aggr_sh.at[idx_v.at[pl.ds(soff + o, cc)]],
                    sems[b], add=True)

        def drain_add(si, b):
            soff = si * SB
            for o, cc in sub:
                pltpu.make_async_copy(
                    rows_v.at[b].at[pl.ds(o, cc)],
                    aggr_sh.at[idx_v.at[pl.ds(soff + o, cc)]],
                    sems[b]).wait()

        pltpu.sync_copy(msg_hbm.at[pl.ds(base, SB)], rows_v.at[0])

        def outer(oi, _):
            for b in range(2):
                si = 2 * oi + b
                issue_add(si, b)

                @pl.when(si + 1 < n_sup)
                def _():
                    pltpu.sync_copy(
                        msg_hbm.at[pl.ds(base + (si + 1) * SB, SB)],
                        rows_v.at[1 - b])

                drain_add(si, b)
            return _

        lax.fori_loop(0, n_sup // 2, outer, None)
        if n_sup % 2:
            issue_add(n_sup - 1, 0)
            drain_add(n_sup - 1, 0)
        plsc.subcore_barrier()

        @pl.when(s == 0)
        def _():
            pltpu.sync_copy(aggr_sh, out_hbm.at[c])

    return scatter_k


def _edge_mlp_body(xi_ref, xj_ref, ea_ref,
                   w1a_ref, w1b_ref, w1c_ref, b1_ref,
                   w2_ref, b2_ref, w3_ref, b3_ref, w4_ref, b4_ref,
                   out_ref):
    bf = jnp.bfloat16
    h = (jnp.dot(xi_ref[...].astype(bf), w1a_ref[...], preferred_element_type=jnp.float32)
         + jnp.dot(xj_ref[...].astype(bf), w1b_ref[...], preferred_element_type=jnp.float32)
         + jnp.dot(ea_ref[...], w1c_ref[...], preferred_element_type=jnp.float32)
         + b1_ref[...])
    h = jnp.maximum(h, 0.0).astype(bf)
    h = jnp.maximum(
        jnp.dot(h, w2_ref[...], preferred_element_type=jnp.float32) + b2_ref[...], 0.0).astype(bf)
    h = jnp.maximum(
        jnp.dot(h, w3_ref[...], preferred_element_type=jnp.float32) + b3_ref[...], 0.0).astype(bf)
    out_ref[...] = (
        jnp.dot(h, w4_ref[...], preferred_element_type=jnp.float32) + b4_ref[...])


def _run_edge_mlp(xi, xj, ea, w1a, w1b, w1c, b1, w2, b2, w3, b3, w4, b4, BE):
    E, D = xi.shape
    DE = ea.shape[1]
    H = w2.shape[0]
    M = w4.shape[1]
    nblk = E // BE
    full = lambda shape: pl.BlockSpec(shape, lambda i: (0,) * len(shape))
    return pl.pallas_call(
        _edge_mlp_body,
        grid=(nblk,),
        in_specs=[
            pl.BlockSpec((BE, D), lambda i: (i, 0)),
            pl.BlockSpec((BE, D), lambda i: (i, 0)),
            pl.BlockSpec((BE, DE), lambda i: (i, 0)),
            full((D, H)), full((D, H)), full((DE, H)), full((1, H)),
            full((H, H)), full((1, H)),
            full((H, H)), full((1, H)),
            full((H, M)), full((1, M)),
        ],
        out_specs=pl.BlockSpec((BE, M), lambda i: (i, 0)),
        out_shape=jax.ShapeDtypeStruct((E, M), jnp.float32),
    )(xi, xj, ea, w1a, w1b, w1c, b1, w2, b2, w3, b3, w4, b4)


def _node_pool_body(x_ref, ap_ref, batch_ref,
                    v1a_ref, v1b_ref, c1_ref, v2_ref, c2_ref,
                    v3_ref, c3_ref, v4_ref, c4_ref, wl_ref, bl_ref,
                    out_ref, sum_acc, cnt_acc, *, nblk, n_graphs):
    i = pl.program_id(0)

    @pl.when(i == 0)
    def _():
        sum_acc[...] = jnp.zeros_like(sum_acc)
        cnt_acc[...] = jnp.zeros_like(cnt_acc)

    bf = jnp.bfloat16
    aggr = (ap_ref[0] + ap_ref[1]).astype(bf)
    h = (jnp.dot(x_ref[...].astype(bf), v1a_ref[...], preferred_element_type=jnp.float32)
         + jnp.dot(aggr, v1b_ref[...], preferred_element_type=jnp.float32)
         + c1_ref[...])
    h = jnp.maximum(h, 0.0).astype(bf)
    h = jnp.maximum(
        jnp.dot(h, v2_ref[...], preferred_element_type=jnp.float32) + c2_ref[...], 0.0).astype(bf)
    h = jnp.maximum(
        jnp.dot(h, v3_ref[...], preferred_element_type=jnp.float32) + c3_ref[...], 0.0).astype(bf)
    node = (jnp.dot(h, v4_ref[...], preferred_element_type=jnp.float32)
            + c4_ref[...])

    b = batch_ref[0]
    gids = lax.broadcasted_iota(jnp.int32, (n_graphs, b.shape[1]), 0)
    mask = (gids == b).astype(jnp.float32)
    sum_acc[...] += jnp.dot(mask, node, preferred_element_type=jnp.float32)
    cnt_acc[...] += jnp.sum(mask, axis=1, keepdims=True)

    @pl.when(i == nblk - 1)
    def _():
        pooled = sum_acc[...] / jnp.maximum(cnt_acc[...], 1.0)
        out_ref[...] = (
            jnp.dot(pooled, wl_ref[...], preferred_element_type=jnp.float32)
            + bl_ref[...])


def _run_node_pool(x, aggr_p, batch3, v1a, v1b, c1, v2, c2, v3, c3, v4, c4,
                   wl, bl, BN, n_graphs):
    N, D = x.shape
    H = v2.shape[0]
    NH = v4.shape[1]
    P = wl.shape[1]
    nblk = N // BN
    full = lambda shape: pl.BlockSpec(shape, lambda i: (0,) * len(shape))
    body = functools.partial(_node_pool_body, nblk=nblk, n_graphs=n_graphs)
    return pl.pallas_call(
        body,
        grid=(nblk,),
        in_specs=[
            pl.BlockSpec((BN, D), lambda i: (i, 0)),
            pl.BlockSpec((NC, BN, D), lambda i: (0, i, 0)),
            pl.BlockSpec((1, 1, BN), lambda i: (i, 0, 0)),
            full((D, H)), full((D, H)), full((1, H)),
            full((H, H)), full((1, H)),
            full((H, H)), full((1, H)),
            full((H, NH)), full((1, NH)),
            full((NH, P)), full((1, P)),
        ],
        out_specs=pl.BlockSpec((n_graphs, P), lambda i: (0, 0)),
        out_shape=jax.ShapeDtypeStruct((n_graphs, P), jnp.float32),
        scratch_shapes=[
            pltpu.VMEM((n_graphs, NH), jnp.float32),
            pltpu.VMEM((n_graphs, 1), jnp.float32),
        ],
    )(x, aggr_p, batch3, v1a, v1b, c1, v2, c2, v3, c3, v4, c4, wl, bl)


def kernel(x, edge_index, edge_attr, batch,
           W1, b1, W2, b2, W3, b3, W4, b4,
           V1, c1, V2, c2, V3, c3, V4, c4,
           Wl, bl):
    N, D = x.shape
    E = edge_index.shape[1]
    DE = edge_attr.shape[1]
    N_GRAPHS = 64
    SB = 200
    BE = 2560
    BN = 2000

    src = edge_index[0]
    dst = edge_index[1]

    xi, xj = _make_gather(E, N, D, SB, jnp.float32)(x, dst, src)

    bf = jnp.bfloat16
    w1a, w1b, w1c = W1[:D].astype(bf), W1[D:2 * D].astype(bf), W1[2 * D:].astype(bf)
    msg = _run_edge_mlp(
        xi, xj, edge_attr.astype(bf),
        w1a, w1b, w1c, b1.reshape(1, -1),
        W2.astype(bf), b2.reshape(1, -1), W3.astype(bf), b3.reshape(1, -1),
        W4.astype(bf), b4.reshape(1, -1),
        BE)

    zeros = jnp.zeros((N, D), jnp.float32)
    aggr_p = _make_scatter(E, N, D, 80)(msg, dst, zeros)

    batch3 = batch.reshape(N // BN, 1, BN)
    v1a, v1b = V1[:D].astype(bf), V1[D:].astype(bf)
    out = _run_node_pool(
        x, aggr_p, batch3,
        v1a, v1b, c1.reshape(1, -1),
        V2.astype(bf), c2.reshape(1, -1), V3.astype(bf), c3.reshape(1, -1),
        V4.astype(bf), c4.reshape(1, -1),
        Wl, bl.reshape(1, -1),
        BN, N_GRAPHS)
    return out


# speedup vs baseline: 1.1703x; 1.0172x over previous
import functools

import jax
import jax.numpy as jnp
from jax import lax
from jax.experimental import pallas as pl
from jax.experimental.pallas import tpu as pltpu
from jax.experimental.pallas import tpu_sc as plsc

NC, NS = 2, 16
NW = NC * NS


def _sc_mesh():
    return plsc.VectorSubcoreMesh(
        core_axis_name="c", subcore_axis_name="s", num_cores=NC, num_subcores=NS
    )


def _make_gather(E, N, D, SB, dtype):
    EW = E // NW
    n_sup = EW // SB
    assert EW % SB == 0 and SB % 8 == 0
    sub = []
    off = 0
    while off < SB:
        c = min(120, SB - off)
        sub.append((off, c))
        off += c
    assert all(c % 8 == 0 and o % 8 == 0 for o, c in sub)

    @functools.partial(
        pl.kernel,
        out_type=(
            jax.ShapeDtypeStruct((E, D), dtype),
            jax.ShapeDtypeStruct((E, D), dtype),
        ),
        mesh=_sc_mesh(),
        scratch_types=[
            pltpu.VMEM((EW,), jnp.int32),
            pltpu.VMEM((EW,), jnp.int32),
            pltpu.VMEM((2, SB, D), dtype),
            pltpu.VMEM((2, SB, D), dtype),
            pltpu.SemaphoreType.DMA,
            pltpu.SemaphoreType.DMA,
        ],
    )
    def gather_k(x_hbm, dst_hbm, src_hbm, xi_hbm, xj_hbm,
                 idx_i, idx_j, rows_i, rows_j, sem0, sem1):
        wid = lax.axis_index("s") * NC + lax.axis_index("c")
        base = wid * EW
        pltpu.sync_copy(dst_hbm.at[pl.ds(base, EW)], idx_i)
        pltpu.sync_copy(src_hbm.at[pl.ds(base, EW)], idx_j)
        sems = (sem0, sem1)

        def issue(si, b):
            soff = si * SB
            for o, c in sub:
                pltpu.async_copy(
                    x_hbm.at[idx_i.at[pl.ds(soff + o, c)]],
                    rows_i.at[b].at[pl.ds(o, c)], sems[b])
                pltpu.async_copy(
                    x_hbm.at[idx_j.at[pl.ds(soff + o, c)]],
                    rows_j.at[b].at[pl.ds(o, c)], sems[b])

        def drain(si, b):
            soff = si * SB
            for o, c in sub:
                pltpu.make_async_copy(
                    x_hbm.at[idx_i.at[pl.ds(soff + o, c)]],
                    rows_i.at[b].at[pl.ds(o, c)], sems[b]).wait()
                pltpu.make_async_copy(
                    x_hbm.at[idx_j.at[pl.ds(soff + o, c)]],
                    rows_j.at[b].at[pl.ds(o, c)], sems[b]).wait()

        issue(0, 0)

        def outer(oi, _):
            for b in range(2):
                si = 2 * oi + b

                @pl.when(si + 1 < n_sup)
                def _():
                    issue(si + 1, 1 - b)

                drain(si, b)
                pltpu.sync_copy(rows_i.at[b], xi_hbm.at[pl.ds(base + si * SB, SB)])
                pltpu.sync_copy(rows_j.at[b], xj_hbm.at[pl.ds(base + si * SB, SB)])
            return _

        lax.fori_loop(0, n_sup // 2, outer, None)

    return gather_k


def _make_scatter(Eg, G, N, D, SB):
    EW = Eg // NW
    n_sup = EW // SB
    assert EW % SB == 0 and SB % 8 == 0
    sub = []
    off = 0
    while off < SB:
        c = min(120, SB - off)
        sub.append((off, c))
        off += c

    @functools.partial(
        pl.kernel,
        out_type=jax.ShapeDtypeStruct((NC, N, D), jnp.float32),
        mesh=_sc_mesh(),
        scratch_types=[
            pltpu.VMEM((EW,), jnp.int32),
            pltpu.VMEM((2, SB, D), jnp.float32),
            pltpu.VMEM_SHARED((N, D), jnp.float32),
            pltpu.SemaphoreType.DMA,
            pltpu.SemaphoreType.DMA,
        ],
    )
    def scatter_k(*refs):
        msgs = refs[:G]
        dst_hbm, zeros_hbm, out_hbm = refs[G], refs[G + 1], refs[G + 2]
        idx_v, rows_v, aggr_sh, sem0, sem1 = refs[G + 3:]
        c = lax.axis_index("c")
        s = lax.axis_index("s")
        wid = s * NC + c
        sems = (sem0, sem1)

        @pl.when(s == 0)
        def _():
            pltpu.sync_copy(zeros_hbm, aggr_sh)

        plsc.subcore_barrier()

        def issue_add(si, b):
            soff = si * SB
            for o, cc in sub:
                pltpu.async_copy(
                    rows_v.at[b].at[pl.ds(o, cc)],
                    aggr_sh.at[idx_v.at[pl.ds(soff + o, cc)]],
                    sems[b], add=True)

        def drain_add(si, b):
            soff = si * SB
            for o, cc in sub:
                pltpu.make_async_copy(
                    rows_v.at[b].at[pl.ds(o, cc)],
                    aggr_sh.at[idx_v.at[pl.ds(soff + o, cc)]],
                    sems[b]).wait()

        for g in range(G):
            msg_hbm = msgs[g]
            base = wid * EW
            pltpu.sync_copy(dst_hbm.at[pl.ds(g * Eg + base, EW)], idx_v)
            pltpu.sync_copy(msg_hbm.at[pl.ds(base, SB)], rows_v.at[0])

            def outer(oi, _, msg_hbm=msg_hbm, base=base):
                for b in range(2):
                    si = 2 * oi + b
                    issue_add(si, b)

                    @pl.when(si + 1 < n_sup)
                    def _():
                        pltpu.sync_copy(
                            msg_hbm.at[pl.ds(base + (si + 1) * SB, SB)],
                            rows_v.at[1 - b])

                    drain_add(si, b)
                return _

            lax.fori_loop(0, n_sup // 2, outer, None)
            if n_sup % 2:
                issue_add(n_sup - 1, 0)
                drain_add(n_sup - 1, 0)

        plsc.subcore_barrier()

        @pl.when(s == 0)
        def _():
            pltpu.sync_copy(aggr_sh, out_hbm.at[c])

    return scatter_k


def _edge_mlp_body(xi_ref, xj_ref, ea_ref,
                   w1a_ref, w1b_ref, w1c_ref, b1_ref,
                   w2_ref, b2_ref, w3_ref, b3_ref, w4_ref, b4_ref,
                   out_ref):
    bf = jnp.bfloat16
    h = (jnp.dot(xi_ref[...].astype(bf), w1a_ref[...], preferred_element_type=jnp.float32)
         + jnp.dot(xj_ref[...].astype(bf), w1b_ref[...], preferred_element_type=jnp.float32)
         + jnp.dot(ea_ref[...], w1c_ref[...], preferred_element_type=jnp.float32)
         + b1_ref[...])
    h = jnp.maximum(h, 0.0).astype(bf)
    h = jnp.maximum(
        jnp.dot(h, w2_ref[...], preferred_element_type=jnp.float32) + b2_ref[...], 0.0).astype(bf)
    h = jnp.maximum(
        jnp.dot(h, w3_ref[...], preferred_element_type=jnp.float32) + b3_ref[...], 0.0).astype(bf)
    out_ref[...] = (
        jnp.dot(h, w4_ref[...], preferred_element_type=jnp.float32) + b4_ref[...])


def _run_edge_mlp(xi, xj, ea, w1a, w1b, w1c, b1, w2, b2, w3, b3, w4, b4, BE):
    E, D = xi.shape
    DE = ea.shape[1]
    H = w2.shape[0]
    M = w4.shape[1]
    nblk = E // BE
    full = lambda shape: pl.BlockSpec(shape, lambda i: (0,) * len(shape))
    return pl.pallas_call(
        _edge_mlp_body,
        grid=(nblk,),
        in_specs=[
            pl.BlockSpec((BE, D), lambda i: (i, 0)),
            pl.BlockSpec((BE, D), lambda i: (i, 0)),
            pl.BlockSpec((BE, DE), lambda i: (i, 0)),
            full((D, H)), full((D, H)), full((DE, H)), full((1, H)),
            full((H, H)), full((1, H)),
            full((H, H)), full((1, H)),
            full((H, M)), full((1, M)),
        ],
        out_specs=pl.BlockSpec((BE, M), lambda i: (i, 0)),
        out_shape=jax.ShapeDtypeStruct((E, M), jnp.float32),
    )(xi, xj, ea, w1a, w1b, w1c, b1, w2, b2, w3, b3, w4, b4)


def _node_pool_body(x_ref, ap_ref, batch_ref,
                    v1a_ref, v1b_ref, c1_ref, v2_ref, c2_ref,
                    v3_ref, c3_ref, v4_ref, c4_ref, wl_ref, bl_ref,
                    out_ref, sum_acc, cnt_acc, *, nblk, n_graphs):
    i = pl.program_id(0)

    @pl.when(i == 0)
    def _():
        sum_acc[...] = jnp.zeros_like(sum_acc)
        cnt_acc[...] = jnp.zeros_like(cnt_acc)

    bf = jnp.bfloat16
    aggr = (ap_ref[0] + ap_ref[1]).astype(bf)
    h = (jnp.dot(x_ref[...].astype(bf), v1a_ref[...], preferred_element_type=jnp.float32)
         + jnp.dot(aggr, v1b_ref[...], preferred_element_type=jnp.float32)
         + c1_ref[...])
    h = jnp.maximum(h, 0.0).astype(bf)
    h = jnp.maximum(
        jnp.dot(h, v2_ref[...], preferred_element_type=jnp.float32) + c2_ref[...], 0.0).astype(bf)
    h = jnp.maximum(
        jnp.dot(h, v3_ref[...], preferred_element_type=jnp.float32) + c3_ref[...], 0.0).astype(bf)
    node = (jnp.dot(h, v4_ref[...], preferred_element_type=jnp.float32)
            + c4_ref[...])

    b = batch_ref[0]
    gids = lax.broadcasted_iota(jnp.int32, (n_graphs, b.shape[1]), 0)
    mask = (gids == b).astype(jnp.float32)
    sum_acc[...] += jnp.dot(mask, node, preferred_element_type=jnp.float32)
    cnt_acc[...] += jnp.sum(mask, axis=1, keepdims=True)

    @pl.when(i == nblk - 1)
    def _():
        pooled = sum_acc[...] / jnp.maximum(cnt_acc[...], 1.0)
        out_ref[...] = (
            jnp.dot(pooled, wl_ref[...], preferred_element_type=jnp.float32)
            + bl_ref[...])


def _run_node_pool(x, aggr_p, batch3, v1a, v1b, c1, v2, c2, v3, c3, v4, c4,
                   wl, bl, BN, n_graphs):
    N, D = x.shape
    H = v2.shape[0]
    NH = v4.shape[1]
    P = wl.shape[1]
    nblk = N // BN
    full = lambda shape: pl.BlockSpec(shape, lambda i: (0,) * len(shape))
    body = functools.partial(_node_pool_body, nblk=nblk, n_graphs=n_graphs)
    return pl.pallas_call(
        body,
        grid=(nblk,),
        in_specs=[
            pl.BlockSpec((BN, D), lambda i: (i, 0)),
            pl.BlockSpec((NC, BN, D), lambda i: (0, i, 0)),
            pl.BlockSpec((1, 1, BN), lambda i: (i, 0, 0)),
            full((D, H)), full((D, H)), full((1, H)),
            full((H, H)), full((1, H)),
            full((H, H)), full((1, H)),
            full((H, NH)), full((1, NH)),
            full((NH, P)), full((1, P)),
        ],
        out_specs=pl.BlockSpec((n_graphs, P), lambda i: (0, 0)),
        out_shape=jax.ShapeDtypeStruct((n_graphs, P), jnp.float32),
        scratch_shapes=[
            pltpu.VMEM((n_graphs, NH), jnp.float32),
            pltpu.VMEM((n_graphs, 1), jnp.float32),
        ],
    )(x, aggr_p, batch3, v1a, v1b, c1, v2, c2, v3, c3, v4, c4, wl, bl)


def kernel(x, edge_index, edge_attr, batch,
           W1, b1, W2, b2, W3, b3, W4, b4,
           V1, c1, V2, c2, V3, c3, V4, c4,
           Wl, bl):
    N, D = x.shape
    E = edge_index.shape[1]
    DE = edge_attr.shape[1]
    N_GRAPHS = 64
    G = 5
    Eg = E // G
    SB = 200
    BE = 2560
    BN = 2000

    src = edge_index[0]
    dst = edge_index[1]

    bf = jnp.bfloat16
    w1a, w1b, w1c = W1[:D].astype(bf), W1[D:2 * D].astype(bf), W1[2 * D:].astype(bf)
    W2b, W3b, W4b = W2.astype(bf), W3.astype(bf), W4.astype(bf)
    ea_bf = edge_attr.astype(bf)

    gather_fn = _make_gather(Eg, N, D, SB, jnp.float32)
    gathered = [
        gather_fn(x, lax.slice(dst, (g * Eg,), ((g + 1) * Eg,)),
                  lax.slice(src, (g * Eg,), ((g + 1) * Eg,)))
        for g in range(G)
    ]
    msgs = [
        _run_edge_mlp(
            xi_g, xj_g, lax.slice(ea_bf, (g * Eg, 0), ((g + 1) * Eg, DE)),
            w1a, w1b, w1c, b1.reshape(1, -1),
            W2b, b2.reshape(1, -1), W3b, b3.reshape(1, -1),
            W4b, b4.reshape(1, -1),
            BE)
        for g, (xi_g, xj_g) in enumerate(gathered)
    ]

    zeros = jnp.zeros((N, D), jnp.float32)
    aggr_p = _make_scatter(Eg, G, N, D, 80)(*msgs, dst, zeros)

    batch3 = batch.reshape(N // BN, 1, BN)
    v1a, v1b = V1[:D].astype(bf), V1[D:].astype(bf)
    out = _run_node_pool(
        x, aggr_p, batch3,
        v1a, v1b, c1.reshape(1, -1),
        V2.astype(bf), c2.reshape(1, -1), V3.astype(bf), c3.reshape(1, -1),
        V4.astype(bf), c4.reshape(1, -1),
        Wl, bl.reshape(1, -1),
        BN, N_GRAPHS)
    return out


# speedup vs baseline: 1.3268x; 1.1337x over previous
import functools

import jax
import jax.numpy as jnp
from jax import lax
from jax.experimental import pallas as pl
from jax.experimental.pallas import tpu as pltpu
from jax.experimental.pallas import tpu_sc as plsc

NC, NS = 2, 16
NW = NC * NS


def _sc_mesh():
    return plsc.VectorSubcoreMesh(
        core_axis_name="c", subcore_axis_name="s", num_cores=NC, num_subcores=NS
    )


def _subchunks(SB):
    sub = []
    off = 0
    while off < SB:
        c = min(120, SB - off)
        sub.append((off, c))
        off += c
    assert all(c % 8 == 0 and o % 8 == 0 for o, c in sub)
    return sub


def _make_gather(E, Eg, goff, N, D, SB):
    EW = Eg // NW
    n_sup = EW // SB
    assert EW % SB == 0 and SB % 8 == 0 and n_sup % 2 == 0
    sub = _subchunks(SB)

    @functools.partial(
        pl.kernel,
        out_type=(
            jax.ShapeDtypeStruct((Eg, D), jnp.float32),
            jax.ShapeDtypeStruct((Eg, D), jnp.float32),
        ),
        mesh=_sc_mesh(),
        scratch_types=[
            pltpu.VMEM((EW,), jnp.int32),
            pltpu.VMEM((EW,), jnp.int32),
            pltpu.VMEM((2, SB, D), jnp.float32),
            pltpu.VMEM((2, SB, D), jnp.float32),
            pltpu.SemaphoreType.DMA,
            pltpu.SemaphoreType.DMA,
        ],
    )
    def gather_k(x_hbm, ei_hbm, xi_hbm, xj_hbm,
                 idx_i, idx_j, rows_i, rows_j, sem0, sem1):
        wid = lax.axis_index("s") * NC + lax.axis_index("c")
        base = wid * EW
        pltpu.sync_copy(ei_hbm.at[pl.ds(E + goff + base, EW)], idx_i)
        pltpu.sync_copy(ei_hbm.at[pl.ds(goff + base, EW)], idx_j)
        sems = (sem0, sem1)

        def issue(si, b):
            soff = si * SB
            for o, c in sub:
                pltpu.async_copy(
                    x_hbm.at[idx_i.at[pl.ds(soff + o, c)]],
                    rows_i.at[b].at[pl.ds(o, c)], sems[b])
                pltpu.async_copy(
                    x_hbm.at[idx_j.at[pl.ds(soff + o, c)]],
                    rows_j.at[b].at[pl.ds(o, c)], sems[b])

        def drain(si, b):
            soff = si * SB
            for o, c in sub:
                pltpu.make_async_copy(
                    x_hbm.at[idx_i.at[pl.ds(soff + o, c)]],
                    rows_i.at[b].at[pl.ds(o, c)], sems[b]).wait()
                pltpu.make_async_copy(
                    x_hbm.at[idx_j.at[pl.ds(soff + o, c)]],
                    rows_j.at[b].at[pl.ds(o, c)], sems[b]).wait()

        issue(0, 0)

        def outer(oi, _):
            for b in range(2):
                si = 2 * oi + b

                @pl.when(si + 1 < n_sup)
                def _():
                    issue(si + 1, 1 - b)

                drain(si, b)
                pltpu.sync_copy(rows_i.at[b], xi_hbm.at[pl.ds(base + si * SB, SB)])
                pltpu.sync_copy(rows_j.at[b], xj_hbm.at[pl.ds(base + si * SB, SB)])
            return _

        lax.fori_loop(0, n_sup // 2, outer, None)

    return gather_k


def _make_scatter(E, Eg, goffs, N, D, SB):
    EW = Eg // NW
    n_sup = EW // SB
    assert EW % SB == 0 and SB % 8 == 0
    G = len(goffs)
    sub = _subchunks(SB)

    @functools.partial(
        pl.kernel,
        out_type=jax.ShapeDtypeStruct((NC, N, D), jnp.float32),
        mesh=_sc_mesh(),
        scratch_types=[
            pltpu.VMEM((EW,), jnp.int32),
            pltpu.VMEM((2, SB, D), jnp.float32),
            pltpu.VMEM_SHARED((N, D), jnp.float32),
            pltpu.SemaphoreType.DMA,
            pltpu.SemaphoreType.DMA,
        ],
    )
    def scatter_k(*refs):
        msgs = refs[:G]
        ei_hbm, zeros_hbm, out_hbm = refs[G], refs[G + 1], refs[G + 2]
        idx_v, rows_v, aggr_sh, sem0, sem1 = refs[G + 3:]
        c = lax.axis_index("c")
        s = lax.axis_index("s")
        wid = s * NC + c
        sems = (sem0, sem1)

        @pl.when(s == 0)
        def _():
            pltpu.sync_copy(zeros_hbm, aggr_sh)

        plsc.subcore_barrier()

        def issue_add(si, b):
            soff = si * SB
            for o, cc in sub:
                pltpu.async_copy(
                    rows_v.at[b].at[pl.ds(o, cc)],
                    aggr_sh.at[idx_v.at[pl.ds(soff + o, cc)]],
                    sems[b], add=True)

        def drain_add(si, b):
            soff = si * SB
            for o, cc in sub:
                pltpu.make_async_copy(
                    rows_v.at[b].at[pl.ds(o, cc)],
                    aggr_sh.at[idx_v.at[pl.ds(soff + o, cc)]],
                    sems[b]).wait()

        for g in range(G):
            msg_hbm = msgs[g]
            base = wid * EW
            pltpu.sync_copy(
                ei_hbm.at[pl.ds(E + goffs[g] + base, EW)], idx_v)
            pltpu.sync_copy(msg_hbm.at[pl.ds(base, SB)], rows_v.at[0])

            def outer(oi, _, msg_hbm=msg_hbm, base=base):
                for b in range(2):
                    si = 2 * oi + b
                    issue_add(si, b)

                    @pl.when(si + 1 < n_sup)
                    def _():
                        pltpu.sync_copy(
                            msg_hbm.at[pl.ds(base + (si + 1) * SB, SB)],
                            rows_v.at[1 - b])

                    drain_add(si, b)
                return _

            lax.fori_loop(0, n_sup // 2, outer, None)
            if n_sup % 2:
                issue_add(n_sup - 1, 0)
                drain_add(n_sup - 1, 0)

        plsc.subcore_barrier()

        @pl.when(s == 0)
        def _():
            pltpu.sync_copy(aggr_sh, out_hbm.at[c])

    return scatter_k


def _edge_mlp_body(xi_ref, xj_ref, ea_ref,
                   w1a_ref, w1b_ref, w1c_ref, b1_ref,
                   w2_ref, b2_ref, w3_ref, b3_ref, w4_ref, b4_ref,
                   out_ref):
    h = (jnp.dot(xi_ref[...], w1a_ref[...], preferred_element_type=jnp.float32)
         + jnp.dot(xj_ref[...], w1b_ref[...], preferred_element_type=jnp.float32)
         + jnp.dot(ea_ref[...], w1c_ref[...], preferred_element_type=jnp.float32)
         + b1_ref[...])
    h = jnp.maximum(h, 0.0)
    h = jnp.maximum(
        jnp.dot(h, w2_ref[...], preferred_element_type=jnp.float32) + b2_ref[...], 0.0)
    h = jnp.maximum(
        jnp.dot(h, w3_ref[...], preferred_element_type=jnp.float32) + b3_ref[...], 0.0)
    out_ref[...] = (
        jnp.dot(h, w4_ref[...], preferred_element_type=jnp.float32) + b4_ref[...])


def _run_edge_mlp(xi, xj, ea, gblk,
                  w1a, w1b, w1c, b1, w2, b2, w3, b3, w4, b4, BE):
    Eg, D = xi.shape
    DE = ea.shape[1]
    H = w2.shape[0]
    M = w4.shape[1]
    nblk = Eg // BE
    full = lambda shape: pl.BlockSpec(shape, lambda i: (0,) * len(shape))
    return pl.pallas_call(
        _edge_mlp_body,
        grid=(nblk,),
        in_specs=[
            pl.BlockSpec((BE, D), lambda i: (i, 0)),
            pl.BlockSpec((BE, D), lambda i: (i, 0)),
            pl.BlockSpec((BE, DE), lambda i: (gblk + i, 0)),
            full((D, H)), full((D, H)), full((DE, H)), full((1, H)),
            full((H, H)), full((1, H)),
            full((H, H)), full((1, H)),
            full((H, M)), full((1, M)),
        ],
        out_specs=pl.BlockSpec((BE, M), lambda i: (i, 0)),
        out_shape=jax.ShapeDtypeStruct((Eg, M), jnp.float32),
    )(xi, xj, ea, w1a, w1b, w1c, b1, w2, b2, w3, b3, w4, b4)


def _node_pool_body(x_ref, ap_ref, batch_ref,
                    v1a_ref, v1b_ref, c1_ref, v2_ref, c2_ref,
                    v3_ref, c3_ref, v4_ref, c4_ref, wl_ref, bl_ref,
                    out_ref, sum_acc, cnt_acc, *, nblk, n_graphs, n_part):
    i = pl.program_id(0)

    @pl.when(i == 0)
    def _():
        sum_acc[...] = jnp.zeros_like(sum_acc)
        cnt_acc[...] = jnp.zeros_like(cnt_acc)

    aggr = ap_ref[0]
    for p in range(1, n_part):
        aggr = aggr + ap_ref[p]
    h = (jnp.dot(x_ref[...], v1a_ref[...], preferred_element_type=jnp.float32)
         + jnp.dot(aggr, v1b_ref[...], preferred_element_type=jnp.float32)
         + c1_ref[...])
    h = jnp.maximum(h, 0.0)
    h = jnp.maximum(
        jnp.dot(h, v2_ref[...], preferred_element_type=jnp.float32) + c2_ref[...], 0.0)
    h = jnp.maximum(
        jnp.dot(h, v3_ref[...], preferred_element_type=jnp.float32) + c3_ref[...], 0.0)
    node = (jnp.dot(h, v4_ref[...], preferred_element_type=jnp.float32)
            + c4_ref[...])

    b = batch_ref[0]
    gids = lax.broadcasted_iota(jnp.int32, (n_graphs, b.shape[1]), 0)
    mask = (gids == b).astype(jnp.float32)
    sum_acc[...] += jnp.dot(mask, node, preferred_element_type=jnp.float32)
    cnt_acc[...] += jnp.sum(mask, axis=1, keepdims=True)

    @pl.when(i == nblk - 1)
    def _():
        pooled = sum_acc[...] / jnp.maximum(cnt_acc[...], 1.0)
        out_ref[...] = (
            jnp.dot(pooled, wl_ref[...], preferred_element_type=jnp.float32)
            + bl_ref[...])


def _run_node_pool(x, aggr_ps, batch3, v1a, v1b, c1, v2, c2, v3, c3, v4, c4,
                   wl, bl, BN, n_graphs):
    N, D = x.shape
    H = v2.shape[0]
    NH = v4.shape[1]
    P = wl.shape[1]
    n_part = aggr_ps.shape[0]
    nblk = N // BN
    full = lambda shape: pl.BlockSpec(shape, lambda i: (0,) * len(shape))
    body = functools.partial(_node_pool_body, nblk=nblk, n_graphs=n_graphs,
                             n_part=n_part)
    return pl.pallas_call(
        body,
        grid=(nblk,),
        in_specs=[
            pl.BlockSpec((BN, D), lambda i: (i, 0)),
            pl.BlockSpec((n_part, BN, D), lambda i: (0, i, 0)),
            pl.BlockSpec((1, 1, BN), lambda i: (i, 0, 0)),
            full((D, H)), full((D, H)), full((1, H)),
            full((H, H)), full((1, H)),
            full((H, H)), full((1, H)),
            full((H, NH)), full((1, NH)),
            full((NH, P)), full((1, P)),
        ],
        out_specs=pl.BlockSpec((n_graphs, P), lambda i: (0, 0)),
        out_shape=jax.ShapeDtypeStruct((n_graphs, P), jnp.float32),
        scratch_shapes=[
            pltpu.VMEM((n_graphs, NH), jnp.float32),
            pltpu.VMEM((n_graphs, 1), jnp.float32),
        ],
    )(x, aggr_ps, batch3, v1a, v1b, c1, v2, c2, v3, c3, v4, c4, wl, bl)


def kernel(x, edge_index, edge_attr, batch,
           W1, b1, W2, b2, W3, b3, W4, b4,
           V1, c1, V2, c2, V3, c3, V4, c4,
           Wl, bl):
    N, D = x.shape
    E = edge_index.shape[1]
    DE = edge_attr.shape[1]
    N_GRAPHS = 64
    G = 5
    Eg = E // G
    SB = 200
    BE = 2560
    BN = 2000

    w1a, w1b, w1c = W1[:D], W1[D:2 * D], W1[2 * D:]
    ei_flat = edge_index.reshape(-1)

    gathered = [
        _make_gather(E, Eg, g * Eg, N, D, SB)(x, ei_flat)
        for g in range(G)
    ]
    msgs = [
        _run_edge_mlp(
            xi_g, xj_g, edge_attr, g * (Eg // BE),
            w1a, w1b, w1c, b1.reshape(1, -1),
            W2, b2.reshape(1, -1), W3, b3.reshape(1, -1),
            W4, b4.reshape(1, -1),
            BE)
        for g, (xi_g, xj_g) in enumerate(gathered)
    ]

    zeros = jnp.zeros((N, D), jnp.float32)
    ga, gb = (0, 1, 2), (3, 4)
    aggr_a = _make_scatter(E, Eg, tuple(g * Eg for g in ga), N, D, 80)(
        *[msgs[g] for g in ga], ei_flat, zeros)
    aggr_b = _make_scatter(E, Eg, tuple(g * Eg for g in gb), N, D, 80)(
        *[msgs[g] for g in gb], ei_flat, zeros)
    aggr_ps = jnp.stack([aggr_a[0], aggr_a[1], aggr_b[0], aggr_b[1]])

    batch3 = batch.reshape(N // BN, 1, BN)
    v1a, v1b = V1[:D], V1[D:]
    out = _run_node_pool(
        x, aggr_ps, batch3,
        v1a, v1b, c1.reshape(1, -1),
        V2, c2.reshape(1, -1), V3, c3.reshape(1, -1), V4, c4.reshape(1, -1),
        Wl, bl.reshape(1, -1),
        BN, N_GRAPHS)
    return out


# speedup vs baseline: 1.5514x; 1.1693x over previous
import functools

import jax
import jax.numpy as jnp
from jax import lax
from jax.experimental import pallas as pl
from jax.experimental.pallas import tpu as pltpu
from jax.experimental.pallas import tpu_sc as plsc

NC, NS = 2, 16
NW = NC * NS


def _sc_mesh():
    return plsc.VectorSubcoreMesh(
        core_axis_name="c", subcore_axis_name="s", num_cores=NC, num_subcores=NS
    )


def _subchunks(SB):
    sub = []
    off = 0
    while off < SB:
        c = min(120, SB - off)
        sub.append((off, c))
        off += c
    assert all(c % 8 == 0 and o % 8 == 0 for o, c in sub)
    return sub


def _make_gather(E, Eg, goff, N, D, SB):
    EW = Eg // NW
    n_sup = EW // SB
    assert EW % SB == 0 and SB % 8 == 0 and n_sup % 2 == 0
    sub = _subchunks(SB)

    @functools.partial(
        pl.kernel,
        out_type=(
            jax.ShapeDtypeStruct((Eg, D), jnp.float32),
            jax.ShapeDtypeStruct((Eg, D), jnp.float32),
        ),
        mesh=_sc_mesh(),
        scratch_types=[
            pltpu.VMEM((EW,), jnp.int32),
            pltpu.VMEM((EW,), jnp.int32),
            pltpu.VMEM((2, SB, D), jnp.float32),
            pltpu.VMEM((2, SB, D), jnp.float32),
            pltpu.SemaphoreType.DMA,
            pltpu.SemaphoreType.DMA,
        ],
    )
    def gather_k(x_hbm, ei_hbm, xi_hbm, xj_hbm,
                 idx_i, idx_j, rows_i, rows_j, sem0, sem1):
        wid = lax.axis_index("s") * NC + lax.axis_index("c")
        base = wid * EW
        pltpu.sync_copy(ei_hbm.at[pl.ds(E + goff + base, EW)], idx_i)
        pltpu.sync_copy(ei_hbm.at[pl.ds(goff + base, EW)], idx_j)
        sems = (sem0, sem1)

        def issue(si, b):
            soff = si * SB
            for o, c in sub:
                pltpu.async_copy(
                    x_hbm.at[idx_i.at[pl.ds(soff + o, c)]],
                    rows_i.at[b].at[pl.ds(o, c)], sems[b])
                pltpu.async_copy(
                    x_hbm.at[idx_j.at[pl.ds(soff + o, c)]],
                    rows_j.at[b].at[pl.ds(o, c)], sems[b])

        def drain(si, b):
            soff = si * SB
            for o, c in sub:
                pltpu.make_async_copy(
                    x_hbm.at[idx_i.at[pl.ds(soff + o, c)]],
                    rows_i.at[b].at[pl.ds(o, c)], sems[b]).wait()
                pltpu.make_async_copy(
                    x_hbm.at[idx_j.at[pl.ds(soff + o, c)]],
                    rows_j.at[b].at[pl.ds(o, c)], sems[b]).wait()

        issue(0, 0)

        def outer(oi, _):
            for b in range(2):
                si = 2 * oi + b

                @pl.when(si + 1 < n_sup)
                def _():
                    issue(si + 1, 1 - b)

                drain(si, b)
                pltpu.sync_copy(rows_i.at[b], xi_hbm.at[pl.ds(base + si * SB, SB)])
                pltpu.sync_copy(rows_j.at[b], xj_hbm.at[pl.ds(base + si * SB, SB)])
            return _

        lax.fori_loop(0, n_sup // 2, outer, None)

    return gather_k


def _make_scatter(E, Eg, goffs, N, D, SB):
    EW = Eg // NW
    n_sup = EW // SB
    assert EW % SB == 0 and SB % 8 == 0
    G = len(goffs)
    sub = _subchunks(SB)

    @functools.partial(
        pl.kernel,
        out_type=jax.ShapeDtypeStruct((NC, N, D), jnp.float32),
        mesh=_sc_mesh(),
        scratch_types=[
            pltpu.VMEM((EW,), jnp.int32),
            pltpu.VMEM((2, SB, D), jnp.float32),
            pltpu.VMEM_SHARED((N, D), jnp.float32),
            pltpu.SemaphoreType.DMA,
            pltpu.SemaphoreType.DMA,
        ],
    )
    def scatter_k(*refs):
        msgs = refs[:G]
        ei_hbm, zeros_hbm, out_hbm = refs[G], refs[G + 1], refs[G + 2]
        idx_v, rows_v, aggr_sh, sem0, sem1 = refs[G + 3:]
        c = lax.axis_index("c")
        s = lax.axis_index("s")
        wid = s * NC + c
        sems = (sem0, sem1)

        @pl.when(s == 0)
        def _():
            pltpu.sync_copy(zeros_hbm, aggr_sh)

        plsc.subcore_barrier()

        def issue_add(si, b):
            soff = si * SB
            for o, cc in sub:
                pltpu.async_copy(
                    rows_v.at[b].at[pl.ds(o, cc)],
                    aggr_sh.at[idx_v.at[pl.ds(soff + o, cc)]],
                    sems[b], add=True)

        def drain_add(si, b):
            soff = si * SB
            for o, cc in sub:
                pltpu.make_async_copy(
                    rows_v.at[b].at[pl.ds(o, cc)],
                    aggr_sh.at[idx_v.at[pl.ds(soff + o, cc)]],
                    sems[b]).wait()

        for g in range(G):
            msg_hbm = msgs[g]
            base = wid * EW
            pltpu.sync_copy(
                ei_hbm.at[pl.ds(E + goffs[g] + base, EW)], idx_v)
            pltpu.sync_copy(msg_hbm.at[pl.ds(base, SB)], rows_v.at[0])

            def outer(oi, _, msg_hbm=msg_hbm, base=base):
                for b in range(2):
                    si = 2 * oi + b
                    issue_add(si, b)

                    @pl.when(si + 1 < n_sup)
                    def _():
                        pltpu.sync_copy(
                            msg_hbm.at[pl.ds(base + (si + 1) * SB, SB)],
                            rows_v.at[1 - b])

                    drain_add(si, b)
                return _

            lax.fori_loop(0, n_sup // 2, outer, None)
            if n_sup % 2:
                issue_add(n_sup - 1, 0)
                drain_add(n_sup - 1, 0)

        plsc.subcore_barrier()

        @pl.when(s == 0)
        def _():
            pltpu.sync_copy(aggr_sh, out_hbm.at[c])

    return scatter_k


def _edge_mlp_body(xi_ref, xj_ref, eat_ref,
                   w1a_ref, w1b_ref, w1c_ref, b1_ref,
                   w2_ref, b2_ref, w3_ref, b3_ref, w4_ref, b4_ref,
                   out_ref):
    h = (jnp.dot(xi_ref[...], w1a_ref[...], preferred_element_type=jnp.float32)
         + jnp.dot(xj_ref[...], w1b_ref[...], preferred_element_type=jnp.float32)
         + lax.dot_general(eat_ref[...], w1c_ref[...],
                           (((0,), (0,)), ((), ())),
                           preferred_element_type=jnp.float32)
         + b1_ref[...])
    h = jnp.maximum(h, 0.0)
    h = jnp.maximum(
        jnp.dot(h, w2_ref[...], preferred_element_type=jnp.float32) + b2_ref[...], 0.0)
    h = jnp.maximum(
        jnp.dot(h, w3_ref[...], preferred_element_type=jnp.float32) + b3_ref[...], 0.0)
    out_ref[...] = (
        jnp.dot(h, w4_ref[...], preferred_element_type=jnp.float32) + b4_ref[...])


def _run_edge_mlp(xi, xj, eat, gblk,
                  w1a, w1b, w1c, b1, w2, b2, w3, b3, w4, b4, BE):
    Eg, D = xi.shape
    DE = eat.shape[0]
    H = w2.shape[0]
    M = w4.shape[1]
    nblk = Eg // BE
    full = lambda shape: pl.BlockSpec(shape, lambda i: (0,) * len(shape))
    return pl.pallas_call(
        _edge_mlp_body,
        grid=(nblk,),
        in_specs=[
            pl.BlockSpec((BE, D), lambda i: (i, 0)),
            pl.BlockSpec((BE, D), lambda i: (i, 0)),
            pl.BlockSpec((DE, BE), lambda i: (0, gblk + i)),
            full((D, H)), full((D, H)), full((DE, H)), full((1, H)),
            full((H, H)), full((1, H)),
            full((H, H)), full((1, H)),
            full((H, M)), full((1, M)),
        ],
        out_specs=pl.BlockSpec((BE, M), lambda i: (i, 0)),
        out_shape=jax.ShapeDtypeStruct((Eg, M), jnp.float32),
    )(xi, xj, eat, w1a, w1b, w1c, b1, w2, b2, w3, b3, w4, b4)


def _node_pool_body(x_ref, apa_ref, apb_ref, batch_ref,
                    v1a_ref, v1b_ref, c1_ref, v2_ref, c2_ref,
                    v3_ref, c3_ref, v4_ref, c4_ref, wl_ref, bl_ref,
                    out_ref, sum_acc, cnt_acc, *, nblk, n_graphs):
    i = pl.program_id(0)

    @pl.when(i == 0)
    def _():
        sum_acc[...] = jnp.zeros_like(sum_acc)
        cnt_acc[...] = jnp.zeros_like(cnt_acc)

    aggr = (apa_ref[0] + apa_ref[1]) + (apb_ref[0] + apb_ref[1])
    h = (jnp.dot(x_ref[...], v1a_ref[...], preferred_element_type=jnp.float32)
         + jnp.dot(aggr, v1b_ref[...], preferred_element_type=jnp.float32)
         + c1_ref[...])
    h = jnp.maximum(h, 0.0)
    h = jnp.maximum(
        jnp.dot(h, v2_ref[...], preferred_element_type=jnp.float32) + c2_ref[...], 0.0)
    h = jnp.maximum(
        jnp.dot(h, v3_ref[...], preferred_element_type=jnp.float32) + c3_ref[...], 0.0)
    node = (jnp.dot(h, v4_ref[...], preferred_element_type=jnp.float32)
            + c4_ref[...])

    b = batch_ref[0]
    gids = lax.broadcasted_iota(jnp.int32, (n_graphs, b.shape[1]), 0)
    mask = (gids == b).astype(jnp.float32)
    sum_acc[...] += jnp.dot(mask, node, preferred_element_type=jnp.float32)
    cnt_acc[...] += jnp.sum(mask, axis=1, keepdims=True)

    @pl.when(i == nblk - 1)
    def _():
        pooled = sum_acc[...] / jnp.maximum(cnt_acc[...], 1.0)
        out_ref[...] = (
            jnp.dot(pooled, wl_ref[...], preferred_element_type=jnp.float32)
            + bl_ref[...])


def _run_node_pool(x, aggr_a, aggr_b, batch3, v1a, v1b, c1, v2, c2, v3, c3,
                   v4, c4, wl, bl, BN, n_graphs):
    N, D = x.shape
    H = v2.shape[0]
    NH = v4.shape[1]
    P = wl.shape[1]
    nblk = N // BN
    full = lambda shape: pl.BlockSpec(shape, lambda i: (0,) * len(shape))
    body = functools.partial(_node_pool_body, nblk=nblk, n_graphs=n_graphs)
    return pl.pallas_call(
        body,
        grid=(nblk,),
        in_specs=[
            pl.BlockSpec((BN, D), lambda i: (i, 0)),
            pl.BlockSpec((NC, BN, D), lambda i: (0, i, 0)),
            pl.BlockSpec((NC, BN, D), lambda i: (0, i, 0)),
            pl.BlockSpec((1, 1, BN), lambda i: (i, 0, 0)),
            full((D, H)), full((D, H)), full((1, H)),
            full((H, H)), full((1, H)),
            full((H, H)), full((1, H)),
            full((H, NH)), full((1, NH)),
            full((NH, P)), full((1, P)),
        ],
        out_specs=pl.BlockSpec((n_graphs, P), lambda i: (0, 0)),
        out_shape=jax.ShapeDtypeStruct((n_graphs, P), jnp.float32),
        scratch_shapes=[
            pltpu.VMEM((n_graphs, NH), jnp.float32),
            pltpu.VMEM((n_graphs, 1), jnp.float32),
        ],
    )(x, aggr_a, aggr_b, batch3, v1a, v1b, c1, v2, c2, v3, c3, v4, c4,
      wl, bl)


def kernel(x, edge_index, edge_attr, batch,
           W1, b1, W2, b2, W3, b3, W4, b4,
           V1, c1, V2, c2, V3, c3, V4, c4,
           Wl, bl):
    N, D = x.shape
    E = edge_index.shape[1]
    DE = edge_attr.shape[1]
    N_GRAPHS = 64
    G = 5
    Eg = E // G
    SB = 200
    BE = 2560
    BN = 2000

    w1a, w1b, w1c = W1[:D], W1[D:2 * D], W1[2 * D:]
    ei_flat = edge_index.reshape(-1)

    gathered = [
        _make_gather(E, Eg, g * Eg, N, D, SB)(x, ei_flat)
        for g in range(G)
    ]
    eat = edge_attr.T
    msgs = [
        _run_edge_mlp(
            xi_g, xj_g, eat, g * (Eg // BE),
            w1a, w1b, w1c, b1.reshape(1, -1),
            W2, b2.reshape(1, -1), W3, b3.reshape(1, -1),
            W4, b4.reshape(1, -1),
            BE)
        for g, (xi_g, xj_g) in enumerate(gathered)
    ]

    zeros = jnp.zeros((N, D), jnp.float32)
    ga, gb = (0, 1, 2), (3, 4)
    aggr_a = _make_scatter(E, Eg, tuple(g * Eg for g in ga), N, D, 80)(
        *[msgs[g] for g in ga], ei_flat, zeros)
    aggr_b = _make_scatter(E, Eg, tuple(g * Eg for g in gb), N, D, 80)(
        *[msgs[g] for g in gb], ei_flat, zeros)

    batch3 = batch.reshape(N // BN, 1, BN)
    v1a, v1b = V1[:D], V1[D:]
    out = _run_node_pool(
        x, aggr_a, aggr_b, batch3,
        v1a, v1b, c1.reshape(1, -1),
        V2, c2.reshape(1, -1), V3, c3.reshape(1, -1), V4, c4.reshape(1, -1),
        Wl, bl.reshape(1, -1),
        BN, N_GRAPHS)
    return out
